# Initial kernel scaffold; baseline (speedup 1.0000x reference)
import jax
import jax.numpy as jnp
from jax.experimental import pallas as pl


def kernel(t, emb, times, edge_index_list, edge_type_list, rel1, W1, Wloop1, rel2, W2, Wloop2, res1, res2):
    raise NotImplementedError("write your pallas kernel here")


# speedup vs baseline: 3.1058x; 3.1058x over previous
import jax
import jax.numpy as jnp
from jax import lax
from jax.experimental import pallas as pl
from jax.experimental.pallas import tpu as pltpu
from jax.experimental.pallas import tpu_sc as plsc

N = 10000
E = 160000
D = 256
R = 200
T = 4
H = 128
NC = 2
NS = 16
LANES = 16
EPT = E // NS
C = 80
NCH = EPT // C
NP = 10240
RPT = NP // NS
DEGW = 16


def _make_seg():
  mesh = plsc.VectorSubcoreMesh(core_axis_name="c", subcore_axis_name="s",
                                num_cores=NC, num_subcores=NS)

  def body(xcat, relcat, idx_pack, z_agg, agg_out,
           ibuf, xbuf, rbuf, sh_agg, sem0, sem1):
    c = lax.axis_index("c")
    s = lax.axis_index("s")

    rs = pl.ds(s * RPT, RPT)
    pltpu.sync_copy(z_agg.at[rs], sh_agg.at[rs])
    plsc.subcore_barrier()

    def chunk(j, _):
      pltpu.sync_copy(idx_pack.at[c].at[s].at[j], ibuf)
      cp0 = pltpu.async_copy(xcat.at[ibuf.at[0]], xbuf, sem0)
      cp1 = pltpu.async_copy(relcat.at[ibuf.at[1]], rbuf, sem1)
      cp0.wait()
      cp1.wait()

      def mul_row(i, _):
        for k in range(H // LANES):
          sl = pl.ds(k * LANES, LANES)
          xbuf[i, sl] = xbuf[i, sl] * rbuf[i, sl]
        return 0

      lax.fori_loop(0, C, mul_row, 0)
      pltpu.sync_copy(xbuf, sh_agg.at[ibuf.at[2]], add=True)
      return 0

    lax.fori_loop(0, NCH, chunk, 0)
    plsc.subcore_barrier()
    pltpu.sync_copy(sh_agg.at[rs], agg_out.at[pl.ds(c * NP + s * RPT, RPT)])

  return pl.kernel(
      body,
      out_type=[jax.ShapeDtypeStruct((NC * NP, H), jnp.float32)],
      mesh=mesh,
      scratch_types=[
          pltpu.VMEM((3, C), jnp.int32),
          pltpu.VMEM((C, H), jnp.float32),
          pltpu.VMEM((C, H), jnp.float32),
          pltpu.VMEM_SHARED((NP, H), jnp.float32),
          pltpu.SemaphoreType.DMA,
          pltpu.SemaphoreType.DMA,
      ])


C2 = 40
NCH2 = E // (NC * NS * C2)


def _make_deg():
  mesh = plsc.VectorSubcoreMesh(core_axis_name="c", subcore_axis_name="s",
                                num_cores=NC, num_subcores=NS)

  def body(dst_d, z_agg, ones_in, deg_out, dbuf, onesb, sh_deg, sem0):
    c = lax.axis_index("c")
    s = lax.axis_index("s")
    rs = pl.ds(s * RPT, RPT)
    pltpu.sync_copy(z_agg.at[rs], sh_deg.at[rs])
    pltpu.sync_copy(ones_in, onesb)
    plsc.subcore_barrier()
    w = c * NS + s

    def chunk(j, _):
      pltpu.sync_copy(dst_d.at[w].at[j], dbuf)
      pltpu.sync_copy(onesb, sh_deg.at[dbuf], add=True)
      return 0

    lax.fori_loop(0, NCH2, chunk, 0)
    plsc.subcore_barrier()
    pltpu.sync_copy(sh_deg.at[rs], deg_out.at[pl.ds(c * NP + s * RPT, RPT)])

  return pl.kernel(
      body,
      out_type=[jax.ShapeDtypeStruct((NC * NP, H), jnp.float32)],
      mesh=mesh,
      scratch_types=[
          pltpu.VMEM((C2,), jnp.int32),
          pltpu.VMEM((C2, H), jnp.float32),
          pltpu.VMEM_SHARED((NP, H), jnp.float32),
          pltpu.SemaphoreType.DMA,
      ])


_RB = 1000


def _make_dense(final):
  def body(agg_ref, deg_ref, x_ref, w_ref, wl_ref, res_ref, out_ref):
    aggc = jnp.concatenate([agg_ref[0], agg_ref[1]], axis=1)
    xc = jnp.concatenate([x_ref[0], x_ref[1]], axis=1)
    deg = deg_ref[0, :, 0:1] + deg_ref[1, :, 0:1]
    norm = 1.0 / jnp.maximum(deg, 1.0)
    y = jnp.dot(aggc * norm, w_ref[...], preferred_element_type=jnp.float32)
    y = y + jnp.dot(xc, wl_ref[...], preferred_element_type=jnp.float32)
    h = xc + res_ref[0, 0] * jnp.tanh(y)
    if final:
      out_ref[...] = h
    else:
      out_ref[0] = h[:, :H]
      out_ref[1] = h[:, H:]

  in_specs = [
      pl.BlockSpec((NC, _RB, H), lambda i: (0, i, 0)),
      pl.BlockSpec((NC, _RB, H), lambda i: (0, i, 0)),
      pl.BlockSpec((NC, _RB, H), lambda i: (0, i, 0)),
      pl.BlockSpec((D, D), lambda i: (0, 0)),
      pl.BlockSpec((D, D), lambda i: (0, 0)),
      pl.BlockSpec(memory_space=pltpu.SMEM),
  ]
  if final:
    out_spec = pl.BlockSpec((_RB, D), lambda i: (i, 0))
    out_shape = jax.ShapeDtypeStruct((N, D), jnp.float32)
  else:
    out_spec = pl.BlockSpec((NC, _RB, H), lambda i: (0, i, 0))
    out_shape = jax.ShapeDtypeStruct((NC, N, H), jnp.float32)
  return pl.pallas_call(body, grid=(N // _RB,), in_specs=in_specs,
                        out_specs=out_spec, out_shape=out_shape)


_seg = _make_seg()
_deg = _make_deg()
_dense_mid = _make_dense(False)
_dense_fin = _make_dense(True)


def kernel(t, emb, times, edge_index_list, edge_type_list,
           rel1, W1, Wloop1, rel2, W2, Wloop2, res1, res2):
  idx = jnp.sum((times <= t).astype(jnp.int32)) - 1
  idx = jnp.clip(idx, 0, T - 1)
  edge_index = lax.dynamic_index_in_dim(edge_index_list, idx, 0, keepdims=False)
  edge_type = lax.dynamic_index_in_dim(edge_type_list, idx, 0, keepdims=False)
  src_r = edge_index[0].reshape(NS, NCH, C)
  dst_r = edge_index[1].reshape(NS, NCH, C)
  et_r = edge_type.reshape(NS, NCH, C)
  packs = [jnp.stack([src_r + c * N, et_r + c * R, dst_r], axis=2)
           for c in range(NC)]
  idx_pack = jnp.stack(packs)
  dst_d = edge_index[1].reshape(NC * NS, NCH2, C2)
  z_agg = jnp.zeros((NP, H), jnp.float32)
  ones_in = jnp.ones((C2, H), jnp.float32)

  embT = jnp.stack([emb[:, :H], emb[:, H:]])
  rel1c = jnp.concatenate([rel1[:, :H], rel1[:, H:]], axis=0)
  rel2c = jnp.concatenate([rel2[:, :H], rel2[:, H:]], axis=0)

  degp = _deg(dst_d, z_agg, ones_in)
  if isinstance(degp, (list, tuple)):
    degp = degp[0]
  degp = degp.reshape(NC, NP, H)

  agg1 = _seg(embT.reshape(NC * N, H), rel1c, idx_pack, z_agg)
  if isinstance(agg1, (list, tuple)):
    agg1 = agg1[0]
  agg1 = agg1.reshape(NC, NP, H)
  h1T = _dense_mid(agg1, degp, embT, W1, Wloop1, jnp.reshape(res1, (1, 1)))

  agg2 = _seg(h1T.reshape(NC * N, H), rel2c, idx_pack, z_agg)
  if isinstance(agg2, (list, tuple)):
    agg2 = agg2[0]
  agg2 = agg2.reshape(NC, NP, H)
  h = _dense_fin(agg2, degp, h1T, W2, Wloop2, jnp.reshape(res2, (1, 1)))
  return h


# speedup vs baseline: 3.2438x; 1.0444x over previous
import jax
import jax.numpy as jnp
from jax import lax
from jax.experimental import pallas as pl
from jax.experimental.pallas import tpu as pltpu
from jax.experimental.pallas import tpu_sc as plsc

N = 10000
E = 160000
D = 256
R = 200
T = 4
H = 128
NC = 2
NS = 16
LANES = 16
EPT = E // NS
C = 80
NCH = 126
NCHP = 128
NP = 10240
RPT = NP // NS
DEGW = 16


def _make_seg():
  mesh = plsc.VectorSubcoreMesh(core_axis_name="c", subcore_axis_name="s",
                                num_cores=NC, num_subcores=NS)

  def body(xcat, relcat, idx_pack, z_agg, agg_out,
           ibuf, xbuf, rbuf, sh_agg, semx0, semx1, semr0, semr1):
    c = lax.axis_index("c")
    s = lax.axis_index("s")

    rs = pl.ds(s * RPT, RPT)
    pltpu.sync_copy(z_agg.at[rs], sh_agg.at[rs])
    plsc.subcore_barrier()

    semx = (semx0, semx1)
    semr = (semr0, semr1)
    ip = idx_pack.at[c].at[s]

    def load(j, b):
      pltpu.sync_copy(ip.at[j], ibuf.at[b])
      pltpu.async_copy(xcat.at[ibuf.at[b].at[0]], xbuf.at[b], semx[b])
      pltpu.async_copy(relcat.at[ibuf.at[b].at[1]], rbuf.at[b], semr[b])

    def wait_slot(b):
      pltpu.make_async_copy(z_agg.at[pl.ds(0, C)], xbuf.at[b], semx[b]).wait()
      pltpu.make_async_copy(z_agg.at[pl.ds(0, C)], rbuf.at[b], semr[b]).wait()

    load(0, 0)
    load(1, 1)

    def step(p, _):
      for b in range(2):
        j = 2 * p + b
        wait_slot(b)

        def mul_row(i, _):
          for k in range(H // LANES):
            sl = pl.ds(k * LANES, LANES)
            xbuf[b, i, sl] = xbuf[b, i, sl] * rbuf[b, i, sl]
          return 0

        lax.fori_loop(0, C, mul_row, 0)
        pltpu.sync_copy(xbuf.at[b], sh_agg.at[ibuf.at[b].at[2]], add=True)
        load(j + 2, b)
      return 0

    lax.fori_loop(0, NCH // 2, step, 0)
    wait_slot(0)
    wait_slot(1)
    plsc.subcore_barrier()
    pltpu.sync_copy(sh_agg.at[rs], agg_out.at[pl.ds(c * NP + s * RPT, RPT)])

  return pl.kernel(
      body,
      out_type=[jax.ShapeDtypeStruct((NC * NP, H), jnp.float32)],
      mesh=mesh,
      scratch_types=[
          pltpu.VMEM((2, 3, C), jnp.int32),
          pltpu.VMEM((2, C, H), jnp.float32),
          pltpu.VMEM((2, C, H), jnp.float32),
          pltpu.VMEM_SHARED((NP, H), jnp.float32),
          pltpu.SemaphoreType.DMA,
          pltpu.SemaphoreType.DMA,
          pltpu.SemaphoreType.DMA,
          pltpu.SemaphoreType.DMA,
      ])


C2 = 40
NCH2 = E // (NC * NS * C2)


def _make_deg():
  mesh = plsc.VectorSubcoreMesh(core_axis_name="c", subcore_axis_name="s",
                                num_cores=NC, num_subcores=NS)

  def body(dst_d, z_agg, ones_in, deg_out, dbuf, onesb, sh_deg, sem0):
    c = lax.axis_index("c")
    s = lax.axis_index("s")
    rs = pl.ds(s * RPT, RPT)
    pltpu.sync_copy(z_agg.at[rs], sh_deg.at[rs])
    pltpu.sync_copy(ones_in, onesb)
    plsc.subcore_barrier()
    w = c * NS + s

    def chunk(j, _):
      pltpu.sync_copy(dst_d.at[w].at[j], dbuf)
      pltpu.sync_copy(onesb, sh_deg.at[dbuf], add=True)
      return 0

    lax.fori_loop(0, NCH2, chunk, 0)
    plsc.subcore_barrier()
    pltpu.sync_copy(sh_deg.at[rs], deg_out.at[pl.ds(c * NP + s * RPT, RPT)])

  return pl.kernel(
      body,
      out_type=[jax.ShapeDtypeStruct((NC * NP, H), jnp.float32)],
      mesh=mesh,
      scratch_types=[
          pltpu.VMEM((C2,), jnp.int32),
          pltpu.VMEM((C2, H), jnp.float32),
          pltpu.VMEM_SHARED((NP, H), jnp.float32),
          pltpu.SemaphoreType.DMA,
      ])


_RB = 1000


def _make_dense(final):
  def body(agg_ref, deg_ref, x_ref, w_ref, wl_ref, res_ref, out_ref):
    aggc = jnp.concatenate([agg_ref[0], agg_ref[1]], axis=1)
    xc = jnp.concatenate([x_ref[0], x_ref[1]], axis=1)
    deg = deg_ref[0, :, 0:1] + deg_ref[1, :, 0:1]
    norm = 1.0 / jnp.maximum(deg, 1.0)
    y = jnp.dot(aggc * norm, w_ref[...], preferred_element_type=jnp.float32)
    y = y + jnp.dot(xc, wl_ref[...], preferred_element_type=jnp.float32)
    h = xc + res_ref[0, 0] * jnp.tanh(y)
    if final:
      out_ref[...] = h
    else:
      out_ref[0] = h[:, :H]
      out_ref[1] = h[:, H:]

  in_specs = [
      pl.BlockSpec((NC, _RB, H), lambda i: (0, i, 0)),
      pl.BlockSpec((NC, _RB, H), lambda i: (0, i, 0)),
      pl.BlockSpec((NC, _RB, H), lambda i: (0, i, 0)),
      pl.BlockSpec((D, D), lambda i: (0, 0)),
      pl.BlockSpec((D, D), lambda i: (0, 0)),
      pl.BlockSpec(memory_space=pltpu.SMEM),
  ]
  if final:
    out_spec = pl.BlockSpec((_RB, D), lambda i: (i, 0))
    out_shape = jax.ShapeDtypeStruct((N, D), jnp.float32)
  else:
    out_spec = pl.BlockSpec((NC, _RB, H), lambda i: (0, i, 0))
    out_shape = jax.ShapeDtypeStruct((NC, N, H), jnp.float32)
  return pl.pallas_call(body, grid=(N // _RB,), in_specs=in_specs,
                        out_specs=out_spec, out_shape=out_shape)


_seg = _make_seg()
_deg = _make_deg()
_dense_mid = _make_dense(False)
_dense_fin = _make_dense(True)


def kernel(t, emb, times, edge_index_list, edge_type_list,
           rel1, W1, Wloop1, rel2, W2, Wloop2, res1, res2):
  idx = jnp.sum((times <= t).astype(jnp.int32)) - 1
  idx = jnp.clip(idx, 0, T - 1)
  edge_index = lax.dynamic_index_in_dim(edge_index_list, idx, 0, keepdims=False)
  edge_type = lax.dynamic_index_in_dim(edge_type_list, idx, 0, keepdims=False)
  pad = NCHP * C - EPT
  src_p = jnp.concatenate(
      [edge_index[0].reshape(NS, EPT), jnp.zeros((NS, pad), jnp.int32)], axis=1)
  dst_p = jnp.concatenate(
      [edge_index[1].reshape(NS, EPT), jnp.full((NS, pad), N, jnp.int32)], axis=1)
  et_p = jnp.concatenate(
      [edge_type.reshape(NS, EPT), jnp.zeros((NS, pad), jnp.int32)], axis=1)
  src_r = src_p.reshape(NS, NCHP, C)
  dst_r = dst_p.reshape(NS, NCHP, C)
  et_r = et_p.reshape(NS, NCHP, C)
  packs = [jnp.stack([src_r + c * N, et_r + c * R, dst_r], axis=2)
           for c in range(NC)]
  idx_pack = jnp.stack(packs)
  dst_d = edge_index[1].reshape(NC * NS, NCH2, C2)
  z_agg = jnp.zeros((NP, H), jnp.float32)
  ones_in = jnp.ones((C2, H), jnp.float32)

  embT = jnp.stack([emb[:, :H], emb[:, H:]])
  rel1c = jnp.concatenate([rel1[:, :H], rel1[:, H:]], axis=0)
  rel2c = jnp.concatenate([rel2[:, :H], rel2[:, H:]], axis=0)

  degp = _deg(dst_d, z_agg, ones_in)
  if isinstance(degp, (list, tuple)):
    degp = degp[0]
  degp = degp.reshape(NC, NP, H)

  agg1 = _seg(embT.reshape(NC * N, H), rel1c, idx_pack, z_agg)
  if isinstance(agg1, (list, tuple)):
    agg1 = agg1[0]
  agg1 = agg1.reshape(NC, NP, H)
  h1T = _dense_mid(agg1, degp, embT, W1, Wloop1, jnp.reshape(res1, (1, 1)))

  agg2 = _seg(h1T.reshape(NC * N, H), rel2c, idx_pack, z_agg)
  if isinstance(agg2, (list, tuple)):
    agg2 = agg2[0]
  agg2 = agg2.reshape(NC, NP, H)
  h = _dense_fin(agg2, degp, h1T, W2, Wloop2, jnp.reshape(res2, (1, 1)))
  return h
